# baseline (speedup 1.0000x reference)
import jax
import jax.numpy as jnp
from jax import lax
from jax.experimental import pallas as pl
from jax.experimental.pallas import tpu as pltpu
from jax.experimental.pallas import tpu_sc as plsc

NC, NS, L = 2, 16, 16
NW = NC * NS
B = 4096
L1 = 20
D = 64
NBG = B // (NW * L)
LH = L1 // 2
ROWS = LH * L
XCH = LH * D * L


def _body(xib, yib, xb, yb, hp, vp, out,
          xiv0, xiv1, idxp0, idxp1, g0, g1, xv0, xv1,
          yidx0, yidx1, vidx0, vidx1, vg0, vg1, yv0, yv1,
          wbuf, outbuf,
          semg0, semg1, semv0, semv1):
    xi_v = (xiv0, xiv1)
    idxp = (idxp0, idxp1)
    G = (g0, g1)
    x_v = (xv0, xv1)
    yidx_v = (yidx0, yidx1)
    vidxp = (vidx0, vidx1)
    VG = (vg0, vg1)
    y_v = (yv0, yv1)
    semg = (semg0, semg1)
    semv = (semv0, semv1)

    wid = lax.axis_index("s") * NC + lax.axis_index("c")
    iota = lax.iota(jnp.int32, L)

    def prep_chunk(bgl, h):
        s = (bgl * 2 + h) % 2
        bg = wid * NBG + bgl
        xi_off = (bg * L1 + h * LH) * L
        pltpu.sync_copy(xib.at[pl.ds(xi_off, ROWS)], xi_v[s])
        for t in range(LH):
            v = xi_v[s][pl.ds(t * L, L)]
            idxp[s][pl.ds(t * L, L)] = lax.shift_right_logical(v, 1)
        half = ROWS // 2
        pltpu.make_async_copy(hp.at[idxp[s].at[pl.ds(0, half)]],
                              G[s].at[pl.ds(0, half)], semg[s]).start()
        pltpu.make_async_copy(hp.at[idxp[s].at[pl.ds(half, half)]],
                              G[s].at[pl.ds(half, half)], semg[s]).start()
        x_off = (bg * L1 + h * LH) * (D * L)
        pltpu.make_async_copy(xb.at[pl.ds(x_off, XCH)], x_v[s],
                              semg[s]).start()

    def prep_bg(bgl):
        s2 = bgl % 2
        bg = wid * NBG + bgl
        pltpu.sync_copy(yib.at[pl.ds(bg * L, L)], yidx_v[s2])
        vidxp[s2][:] = lax.shift_right_logical(yidx_v[s2][:], 1)
        pltpu.make_async_copy(vp.at[vidxp[s2]], VG[s2], semv[s2]).start()
        pltpu.make_async_copy(yb.at[pl.ds(bg * (D * L), D * L)], y_v[s2],
                              semv[s2]).start()

    def compute_w(bgl):
        s2 = bgl % 2
        pltpu.make_async_copy(vp.at[vidxp[s2]], VG[s2], semv[s2]).wait()
        pltpu.make_async_copy(yb.at[pl.ds(0, D * L)], y_v[s2],
                              semv[s2]).wait()
        colbase = (yidx_v[s2][:] & 1) * D

        def dbody(d, carry):
            wv = plsc.load_gather(VG[s2], [iota, colbase + d])
            wbuf[pl.ds(d * L, L)] = wv * y_v[s2][pl.ds(d * L, L)]
            return carry

        lax.fori_loop(0, D, dbody, 0)

    def compute_chunk(bgl, h):
        s = (bgl * 2 + h) % 2
        half = ROWS // 2
        pltpu.make_async_copy(hp.at[idxp[s].at[pl.ds(0, half)]],
                              G[s].at[pl.ds(0, half)], semg[s]).wait()
        pltpu.make_async_copy(hp.at[idxp[s].at[pl.ds(half, half)]],
                              G[s].at[pl.ds(half, half)], semg[s]).wait()
        pltpu.make_async_copy(xb.at[pl.ds(0, XCH)], x_v[s], semg[s]).wait()

        rows = []
        cols = []
        for t in range(LH):
            rows.append(iota + (t * L))
            par = xi_v[s][pl.ds(t * L, L)] & 1
            cols.append(par * D)

        def dbody(d, acc):
            wv = wbuf[pl.ds(d * L, L)]
            part = jnp.zeros((L,), jnp.float32)
            for t in range(LH):
                g = plsc.load_gather(G[s], [rows[t], cols[t] + d])
                xx = x_v[s][pl.ds(t * (D * L) + d * L, L)]
                part = part + g * xx
            return acc + part * wv

        init = jnp.zeros((L,), jnp.float32) if h == 0 else outbuf[:]
        acc = lax.fori_loop(0, D, dbody, init)
        outbuf[:] = acc
        if h == 1:
            bg = wid * NBG + bgl
            pltpu.sync_copy(outbuf, out.at[pl.ds(bg * L, L)])

    prep_bg(0)
    prep_chunk(0, 0)
    for bgl in range(NBG):
        for h in range(2):
            if h == 0:
                prep_chunk(bgl, 1)
            elif bgl + 1 < NBG:
                prep_bg(bgl + 1)
                prep_chunk(bgl + 1, 0)
            if h == 0:
                compute_w(bgl)
            compute_chunk(bgl, h)


@jax.jit
def _gmf_sc(xib, yib, xb, yb, hp, vp):
    mesh = plsc.VectorSubcoreMesh(core_axis_name="c", subcore_axis_name="s")
    scratch = [
        pltpu.VMEM((ROWS,), jnp.int32), pltpu.VMEM((ROWS,), jnp.int32),
        pltpu.VMEM((ROWS,), jnp.int32), pltpu.VMEM((ROWS,), jnp.int32),
        pltpu.VMEM((ROWS, 128), jnp.float32), pltpu.VMEM((ROWS, 128), jnp.float32),
        pltpu.VMEM((XCH,), jnp.float32), pltpu.VMEM((XCH,), jnp.float32),
        pltpu.VMEM((L,), jnp.int32), pltpu.VMEM((L,), jnp.int32),
        pltpu.VMEM((L,), jnp.int32), pltpu.VMEM((L,), jnp.int32),
        pltpu.VMEM((L, 128), jnp.float32), pltpu.VMEM((L, 128), jnp.float32),
        pltpu.VMEM((D * L,), jnp.float32), pltpu.VMEM((D * L,), jnp.float32),
        pltpu.VMEM((D * L,), jnp.float32),
        pltpu.VMEM((L,), jnp.float32),
        pltpu.SemaphoreType.DMA, pltpu.SemaphoreType.DMA,
        pltpu.SemaphoreType.DMA, pltpu.SemaphoreType.DMA,
    ]
    run = pl.kernel(
        _body,
        out_type=jax.ShapeDtypeStruct((B,), jnp.float32),
        mesh=mesh,
        scratch_types=scratch,
        compiler_params=pltpu.CompilerParams(
            needs_layout_passes=False, use_tc_tiling_on_sc=False),
    )
    return run(xib, yib, xb, yb, hp, vp)


def kernel(x_idx, y_idx, x, y, human_table, virus_table):
    ng = B // L
    xib = x_idx.reshape(ng, L, L1).transpose(0, 2, 1).reshape(-1)
    yib = y_idx.reshape(B)
    xb = x.reshape(ng, L, L1, D).transpose(0, 2, 3, 1).reshape(-1)
    yb = y.reshape(ng, L, D).transpose(0, 2, 1).reshape(-1)
    hp = human_table.reshape(500000, 128)
    vp = virus_table.reshape(50000, 128)
    out = _gmf_sc(xib, yib, xb, yb, hp, vp)
    return out.reshape(B, 1)


# speedup vs baseline: 1.1147x; 1.1147x over previous
import jax
import jax.numpy as jnp
from jax import lax
from jax.experimental import pallas as pl
from jax.experimental.pallas import tpu as pltpu
from jax.experimental.pallas import tpu_sc as plsc

NC, NS, L = 2, 16, 16
NW = NC * NS
B = 4096
L1 = 20
D = 64
KD = D // L
BPW = B // NW
C = 16
NCH = BPW // C
RPC = C * L1


def _body(xi, yi, xf, yf, hp, vp, out,
          idx0, idx1, yidx0, yidx1, rows0, rows1, xv0, xv1,
          vrow0, vrow1, yv0, yv1, outv,
          semg0, semg1, semv0, semv1):
    idx_v = (idx0, idx1)
    yidx_v = (yidx0, yidx1)
    rows_v = (rows0, rows1)
    x_v = (xv0, xv1)
    vrow_v = (vrow0, vrow1)
    y_v = (yv0, yv1)
    semg = (semg0, semg1)
    semv = (semv0, semv1)

    wid = lax.axis_index("s") * NC + lax.axis_index("c")
    rbase0 = wid * (BPW * L1)
    bbase0 = wid * BPW
    lane_iota = lax.iota(jnp.int32, L)

    def prep(ch):
        s = ch % 2
        rb = rbase0 + ch * RPC
        bb = bbase0 + ch * C
        pltpu.sync_copy(xi.at[pl.ds(rb, RPC)], idx_v[s].at[pl.ds(0, RPC)])
        pltpu.sync_copy(yi.at[pl.ds(bb, C)], yidx_v[s].at[pl.ds(0, C)])

        def hrow(j, carry):
            r = idx_v[s][pl.ds(j, L)][0]
            pltpu.make_async_copy(hp.at[pl.ds(r * D, D)],
                                  rows_v[s].at[pl.ds(j * D, D)],
                                  semg[s]).start()
            return carry

        lax.fori_loop(0, RPC, hrow, 0)

        def vrow(j, carry):
            r = yidx_v[s][pl.ds(j, L)][0]
            pltpu.make_async_copy(vp.at[pl.ds(r * D, D)],
                                  vrow_v[s].at[pl.ds(j * D, D)],
                                  semv[s]).start()
            return carry

        lax.fori_loop(0, C, vrow, 0)

        pltpu.make_async_copy(xf.at[pl.ds(rb * D, RPC * D)], x_v[s],
                              semg[s]).start()
        pltpu.make_async_copy(yf.at[pl.ds(bb * D, C * D)], y_v[s],
                              semv[s]).start()

    def compute(ch):
        s = ch % 2
        pltpu.make_async_copy(hp.at[pl.ds(0, RPC * D)],
                              rows_v[s], semg[s]).wait()
        pltpu.make_async_copy(xf.at[pl.ds(0, RPC * D)], x_v[s],
                              semg[s]).wait()
        pltpu.make_async_copy(vp.at[pl.ds(0, C * D)],
                              vrow_v[s], semv[s]).wait()
        pltpu.make_async_copy(yf.at[pl.ds(0, C * D)], y_v[s],
                              semv[s]).wait()
        rows = rows_v[s]
        xv = x_v[s]
        vrow = vrow_v[s]
        yv = y_v[s]

        def ebody(e, carry):
            rb = e * L1
            acc = [jnp.zeros((L,), jnp.float32) for _ in range(KD)]
            for l in range(L1):
                for k in range(KD):
                    h = rows[pl.ds((rb + l) * D + k * L, L)]
                    xx = xv[pl.ds((rb + l) * D + k * L, L)]
                    acc[k] = acc[k] + h * xx
            t = jnp.zeros((L,), jnp.float32)
            for k in range(KD):
                w = vrow[pl.ds(e * D + k * L, L)] * yv[pl.ds(e * D + k * L, L)]
                t = t + acc[k] * w
            s_ = jnp.sum(t)
            return jnp.where(lane_iota == e, s_, carry)

        tot = lax.fori_loop(0, C, ebody, jnp.zeros((L,), jnp.float32))
        outv[:] = tot
        bb = bbase0 + ch * C
        pltpu.sync_copy(outv, out.at[pl.ds(bb, C)])

    prep(0)
    for ch in range(NCH):
        if ch + 1 < NCH:
            prep(ch + 1)
        compute(ch)


@jax.jit
def _gmf_sc(xi, yi, xf, yf, hp, vp):
    mesh = plsc.VectorSubcoreMesh(core_axis_name="c", subcore_axis_name="s")
    scratch = [
        pltpu.VMEM((RPC + L,), jnp.int32), pltpu.VMEM((RPC + L,), jnp.int32),
        pltpu.VMEM((C + L,), jnp.int32), pltpu.VMEM((C + L,), jnp.int32),
        pltpu.VMEM((RPC * D,), jnp.float32), pltpu.VMEM((RPC * D,), jnp.float32),
        pltpu.VMEM((RPC * D,), jnp.float32), pltpu.VMEM((RPC * D,), jnp.float32),
        pltpu.VMEM((C * D,), jnp.float32), pltpu.VMEM((C * D,), jnp.float32),
        pltpu.VMEM((C * D,), jnp.float32), pltpu.VMEM((C * D,), jnp.float32),
        pltpu.VMEM((C,), jnp.float32),
        pltpu.SemaphoreType.DMA, pltpu.SemaphoreType.DMA,
        pltpu.SemaphoreType.DMA, pltpu.SemaphoreType.DMA,
    ]
    run = pl.kernel(
        _body,
        out_type=jax.ShapeDtypeStruct((B,), jnp.float32),
        mesh=mesh,
        scratch_types=scratch,
        compiler_params=pltpu.CompilerParams(
            needs_layout_passes=False, use_tc_tiling_on_sc=True),
    )
    return run(xi, yi, xf, yf, hp, vp)


def kernel(x_idx, y_idx, x, y, human_table, virus_table):
    xi = x_idx.reshape(B * L1)
    yi = y_idx.reshape(B)
    xf = x.reshape(B * L1 * D)
    yf = y.reshape(B * D)
    hp = human_table.reshape(1000000 * D)
    vp = virus_table.reshape(100000 * D)
    out = _gmf_sc(xi, yi, xf, yf, hp, vp)
    return out.reshape(B, 1)


# speedup vs baseline: 2.3021x; 2.0651x over previous
import jax
import jax.numpy as jnp
from jax import lax
from jax.experimental import pallas as pl
from jax.experimental.pallas import tpu as pltpu
from jax.experimental.pallas import tpu_sc as plsc

NC, NS, L = 2, 16, 16
NW = NC * NS
B = 4096
L1 = 20
D = 64
KD = D // L
BPW = B // NW
C = 8
NCH = BPW // C
RPC = C * L1
SPC = RPC // 8


def _body(xi, yi, xf, yf, hp, vp, out,
          idx0, idx1, yidx0, yidx1, rows0, rows1, xv0, xv1,
          vrow0, vrow1, yv0, yv1, outv,
          semg0, semg1, semv0, semv1):
    idx_v = (idx0, idx1)
    yidx_v = (yidx0, yidx1)
    rows_v = (rows0, rows1)
    x_v = (xv0, xv1)
    vrow_v = (vrow0, vrow1)
    y_v = (yv0, yv1)
    semg = (semg0, semg1)
    semv = (semv0, semv1)

    wid = lax.axis_index("s") * NC + lax.axis_index("c")
    rbase0 = wid * (BPW * L1)
    bbase0 = wid * BPW
    lane_iota = lax.iota(jnp.int32, L)

    def prep(ch, s):
        rb = rbase0 + ch * RPC
        bb = bbase0 + ch * C
        pltpu.sync_copy(xi.at[pl.ds(rb, RPC)], idx_v[s].at[pl.ds(0, RPC)])
        pltpu.sync_copy(yi.at[pl.ds(bb, C)], yidx_v[s].at[pl.ds(0, C)])

        def hrow(j, carry):
            r = idx_v[s][pl.ds(j, L)][0]
            q = lax.shift_right_logical(r, 3)
            tr = r & 7
            jq = lax.shift_right_logical(j, 3)
            jr = j & 7
            pltpu.make_async_copy(hp.at[pl.ds(q, 1), pl.ds(tr, 1), :],
                                  rows_v[s].at[pl.ds(jq, 1), pl.ds(jr, 1), :],
                                  semg[s]).start()
            return carry

        lax.fori_loop(0, RPC, hrow, 0)

        def vrow(j, carry):
            r = yidx_v[s][pl.ds(j, L)][0]
            q = lax.shift_right_logical(r, 3)
            tr = r & 7
            pltpu.make_async_copy(vp.at[pl.ds(q, 1), pl.ds(tr, 1), :],
                                  vrow_v[s].at[pl.ds(0, 1), pl.ds(j, 1), :],
                                  semv[s]).start()
            return carry

        lax.fori_loop(0, C, vrow, 0)

        pltpu.make_async_copy(xf.at[pl.ds(rb // 8, SPC), :, :], x_v[s],
                              semg[s]).start()
        pltpu.make_async_copy(yf.at[pl.ds(bb // 8, 1), :, :], y_v[s],
                              semv[s]).start()

    def compute(ch, s):
        pltpu.make_async_copy(hp.at[pl.ds(0, SPC), :, :],
                              rows_v[s], semg[s]).wait()
        pltpu.make_async_copy(xf.at[pl.ds(0, SPC), :, :], x_v[s],
                              semg[s]).wait()
        pltpu.make_async_copy(vp.at[pl.ds(0, 1), :, :],
                              vrow_v[s], semv[s]).wait()
        pltpu.make_async_copy(yf.at[pl.ds(0, 1), :, :], y_v[s],
                              semv[s]).wait()
        rows = rows_v[s]
        xv = x_v[s]
        vrow = vrow_v[s]
        yv = y_v[s]

        def ebody(e, carry):
            rb = e * L1
            acc = [jnp.zeros((L,), jnp.float32) for _ in range(KD)]
            for l in range(L1):
                j = rb + l
                jq = lax.shift_right_logical(j, 3)
                jr = j & 7
                for k in range(KD):
                    h = rows[jq, jr, pl.ds(k * L, L)]
                    xx = xv[jq, jr, pl.ds(k * L, L)]
                    acc[k] = acc[k] + h * xx
            t = jnp.zeros((L,), jnp.float32)
            for k in range(KD):
                w = vrow[0, e, pl.ds(k * L, L)] * yv[0, e, pl.ds(k * L, L)]
                t = t + acc[k] * w
            s_ = jnp.sum(t)
            return jnp.where(lane_iota == e, s_, carry)

        tot = lax.fori_loop(0, C, ebody, jnp.zeros((L,), jnp.float32))
        outv[:] = tot
        bb = bbase0 + ch * C
        pltpu.sync_copy(outv.at[pl.ds(0, C)], out.at[pl.ds(bb, C)])

    prep(0, 0)
    prep(1, 1)

    def pair(c2, carry):
        ch0 = c2 * 2
        compute(ch0, 0)
        prep(ch0 + 2, 0)
        compute(ch0 + 1, 1)
        prep(ch0 + 3, 1)
        return carry

    lax.fori_loop(0, NCH // 2 - 1, pair, 0)
    compute(NCH - 2, 0)
    compute(NCH - 1, 1)


@jax.jit
def _gmf_sc(xi, yi, xf, yf, hp, vp):
    mesh = plsc.VectorSubcoreMesh(core_axis_name="c", subcore_axis_name="s")
    scratch = [
        pltpu.VMEM((RPC + L,), jnp.int32), pltpu.VMEM((RPC + L,), jnp.int32),
        pltpu.VMEM((C + L,), jnp.int32), pltpu.VMEM((C + L,), jnp.int32),
        pltpu.VMEM((SPC, 8, D), jnp.float32), pltpu.VMEM((SPC, 8, D), jnp.float32),
        pltpu.VMEM((SPC, 8, D), jnp.float32), pltpu.VMEM((SPC, 8, D), jnp.float32),
        pltpu.VMEM((1, 8, D), jnp.float32), pltpu.VMEM((1, 8, D), jnp.float32),
        pltpu.VMEM((1, 8, D), jnp.float32), pltpu.VMEM((1, 8, D), jnp.float32),
        pltpu.VMEM((L,), jnp.float32),
        pltpu.SemaphoreType.DMA, pltpu.SemaphoreType.DMA,
        pltpu.SemaphoreType.DMA, pltpu.SemaphoreType.DMA,
    ]
    run = pl.kernel(
        _body,
        out_type=jax.ShapeDtypeStruct((B,), jnp.float32),
        mesh=mesh,
        scratch_types=scratch,
        compiler_params=pltpu.CompilerParams(
            needs_layout_passes=False, use_tc_tiling_on_sc=True),
    )
    return run(xi, yi, xf, yf, hp, vp)


def kernel(x_idx, y_idx, x, y, human_table, virus_table):
    xi = x_idx.reshape(B * L1)
    yi = y_idx.reshape(B)
    xf = x.reshape(B * L1 // 8, 8, D)
    yf = y.reshape(B // 8, 8, D)
    hp = human_table.reshape(1000000 // 8, 8, D)
    vp = virus_table.reshape(100000 // 8, 8, D)
    out = _gmf_sc(xi, yi, xf, yf, hp, vp)
    return out.reshape(B, 1)


# speedup vs baseline: 2.3881x; 1.0374x over previous
import jax
import jax.numpy as jnp
from jax import lax
from jax.experimental import pallas as pl
from jax.experimental.pallas import tpu as pltpu
from jax.experimental.pallas import tpu_sc as plsc

NC, NS, L = 2, 16, 16
NW = NC * NS
B = 4096
L1 = 20
D = 64
KD = D // L
BPW = B // NW
C = 8
NCH = BPW // C
RPC = C * L1
SPC = RPC // 8


def _body(xi, yi, xf, yf, hp, vp, out,
          idx0, idx1, yidx0, yidx1, rows0, rows1, xv0, xv1,
          vrow0, vrow1, yv0, yv1, outv,
          semg0, semg1, semv0, semv1, semi0, semi1):
    idx_v = (idx0, idx1)
    yidx_v = (yidx0, yidx1)
    rows_v = (rows0, rows1)
    x_v = (xv0, xv1)
    vrow_v = (vrow0, vrow1)
    y_v = (yv0, yv1)
    semg = (semg0, semg1)
    semv = (semv0, semv1)
    semi = (semi0, semi1)

    wid = lax.axis_index("s") * NC + lax.axis_index("c")
    rbase0 = wid * (BPW * L1)
    bbase0 = wid * BPW
    lane_iota = lax.iota(jnp.int32, L)

    def prep_idx(ch, s):
        rb = rbase0 + ch * RPC
        bb = bbase0 + ch * C
        pltpu.make_async_copy(xi.at[pl.ds(rb, RPC)],
                              idx_v[s].at[pl.ds(0, RPC)], semi[s]).start()
        pltpu.make_async_copy(yi.at[pl.ds(bb, C)],
                              yidx_v[s].at[pl.ds(0, C)], semi[s]).start()

    def prep(ch, s):
        rb = rbase0 + ch * RPC
        bb = bbase0 + ch * C
        pltpu.make_async_copy(xi.at[pl.ds(0, RPC)],
                              idx_v[s].at[pl.ds(0, RPC)], semi[s]).wait()
        pltpu.make_async_copy(yi.at[pl.ds(0, C)],
                              yidx_v[s].at[pl.ds(0, C)], semi[s]).wait()

        def hrow(j, carry):
            r = idx_v[s][pl.ds(j, L)][0]
            q = lax.shift_right_logical(r, 3)
            tr = r & 7
            jq = lax.shift_right_logical(j, 3)
            jr = j & 7
            pltpu.make_async_copy(hp.at[pl.ds(q, 1), pl.ds(tr, 1), :],
                                  rows_v[s].at[pl.ds(jq, 1), pl.ds(jr, 1), :],
                                  semg[s]).start()
            return carry

        lax.fori_loop(0, RPC, hrow, 0)

        def vrow(j, carry):
            r = yidx_v[s][pl.ds(j, L)][0]
            q = lax.shift_right_logical(r, 3)
            tr = r & 7
            pltpu.make_async_copy(vp.at[pl.ds(q, 1), pl.ds(tr, 1), :],
                                  vrow_v[s].at[pl.ds(0, 1), pl.ds(j, 1), :],
                                  semv[s]).start()
            return carry

        lax.fori_loop(0, C, vrow, 0)

        pltpu.make_async_copy(xf.at[pl.ds(rb // 8, SPC), :, :], x_v[s],
                              semg[s]).start()
        pltpu.make_async_copy(yf.at[pl.ds(bb // 8, 1), :, :], y_v[s],
                              semv[s]).start()

    def compute(ch, s):
        pltpu.make_async_copy(hp.at[pl.ds(0, SPC), :, :],
                              rows_v[s], semg[s]).wait()
        pltpu.make_async_copy(xf.at[pl.ds(0, SPC), :, :], x_v[s],
                              semg[s]).wait()
        pltpu.make_async_copy(vp.at[pl.ds(0, 1), :, :],
                              vrow_v[s], semv[s]).wait()
        pltpu.make_async_copy(yf.at[pl.ds(0, 1), :, :], y_v[s],
                              semv[s]).wait()
        rows = rows_v[s]
        xv = x_v[s]
        vrow = vrow_v[s]
        yv = y_v[s]

        def ebody(e, carry):
            rb = e * L1
            acc = [jnp.zeros((L,), jnp.float32) for _ in range(KD)]
            for l in range(L1):
                j = rb + l
                jq = lax.shift_right_logical(j, 3)
                jr = j & 7
                for k in range(KD):
                    h = rows[jq, jr, pl.ds(k * L, L)]
                    xx = xv[jq, jr, pl.ds(k * L, L)]
                    acc[k] = acc[k] + h * xx
            t = jnp.zeros((L,), jnp.float32)
            for k in range(KD):
                w = vrow[0, e, pl.ds(k * L, L)] * yv[0, e, pl.ds(k * L, L)]
                t = t + acc[k] * w
            s_ = jnp.sum(t)
            return jnp.where(lane_iota == e, s_, carry)

        tot = lax.fori_loop(0, C, ebody, jnp.zeros((L,), jnp.float32))
        outv[:] = tot
        bb = bbase0 + ch * C
        pltpu.sync_copy(outv.at[pl.ds(0, C)], out.at[pl.ds(bb, C)])

    prep_idx(0, 0)
    prep_idx(1, 1)
    prep(0, 0)
    prep(1, 1)

    def pair(c2, carry):
        ch0 = c2 * 2
        prep_idx(ch0 + 2, 0)
        compute(ch0, 0)
        prep(ch0 + 2, 0)
        prep_idx(ch0 + 3, 1)
        compute(ch0 + 1, 1)
        prep(ch0 + 3, 1)
        return carry

    lax.fori_loop(0, NCH // 2 - 1, pair, 0)
    compute(NCH - 2, 0)
    compute(NCH - 1, 1)


@jax.jit
def _gmf_sc(xi, yi, xf, yf, hp, vp):
    mesh = plsc.VectorSubcoreMesh(core_axis_name="c", subcore_axis_name="s")
    scratch = [
        pltpu.VMEM((RPC + L,), jnp.int32), pltpu.VMEM((RPC + L,), jnp.int32),
        pltpu.VMEM((C + L,), jnp.int32), pltpu.VMEM((C + L,), jnp.int32),
        pltpu.VMEM((SPC, 8, D), jnp.float32), pltpu.VMEM((SPC, 8, D), jnp.float32),
        pltpu.VMEM((SPC, 8, D), jnp.float32), pltpu.VMEM((SPC, 8, D), jnp.float32),
        pltpu.VMEM((1, 8, D), jnp.float32), pltpu.VMEM((1, 8, D), jnp.float32),
        pltpu.VMEM((1, 8, D), jnp.float32), pltpu.VMEM((1, 8, D), jnp.float32),
        pltpu.VMEM((L,), jnp.float32),
        pltpu.SemaphoreType.DMA, pltpu.SemaphoreType.DMA,
        pltpu.SemaphoreType.DMA, pltpu.SemaphoreType.DMA,
        pltpu.SemaphoreType.DMA, pltpu.SemaphoreType.DMA,
    ]
    run = pl.kernel(
        _body,
        out_type=jax.ShapeDtypeStruct((B,), jnp.float32),
        mesh=mesh,
        scratch_types=scratch,
        compiler_params=pltpu.CompilerParams(
            needs_layout_passes=False, use_tc_tiling_on_sc=True),
    )
    return run(xi, yi, xf, yf, hp, vp)


def kernel(x_idx, y_idx, x, y, human_table, virus_table):
    xi = x_idx.reshape(B * L1)
    yi = y_idx.reshape(B)
    xf = x.reshape(B * L1 // 8, 8, D)
    yf = y.reshape(B // 8, 8, D)
    hp = human_table.reshape(1000000 // 8, 8, D)
    vp = virus_table.reshape(100000 // 8, 8, D)
    out = _gmf_sc(xi, yi, xf, yf, hp, vp)
    return out.reshape(B, 1)
